# baseline (speedup 1.0000x reference)
import functools

import jax
import jax.numpy as jnp
from jax import lax
from jax.experimental import pallas as pl
from jax.experimental.pallas import tpu as pltpu
from jax.experimental.pallas import tpu_sc as plsc

_DIM = 64
_GR = 40
_NBUF = 2


@functools.cache
def _make_kernel(batch, hist):
    B = batch * hist
    info = plsc.get_sparse_core_info()
    nc, ns = info.num_cores, info.num_subcores
    nw = nc * ns
    epw = batch // nw
    bpw = epw * hist
    gpc = hist // _GR
    nouter = epw // _NBUF
    mesh = plsc.VectorSubcoreMesh(core_axis_name="c", subcore_axis_name="s")

    scratch = [pltpu.VMEM((bpw,), jnp.int32)]
    scratch += [pltpu.VMEM((hist, 128), jnp.float32) for _ in range(_NBUF)]
    scratch += [pltpu.VMEM((hist, _DIM), jnp.float32) for _ in range(_NBUF)]
    scratch += [pltpu.SemaphoreType.DMA for _ in range(2 * _NBUF + 1)]

    @functools.partial(
        pl.kernel,
        mesh=mesh,
        out_type=jax.ShapeDtypeStruct((batch, hist, _DIM), jnp.float32),
        scratch_types=scratch,
    )
    def k(idx_hbm, tpad_hbm, out_hbm, idx_v, *bufs_and_sems):
        gbuf = bufs_and_sems[:_NBUF]
        cbuf = bufs_and_sems[_NBUF:2 * _NBUF]
        gsem = bufs_and_sems[2 * _NBUF:3 * _NBUF]
        wsem = bufs_and_sems[3 * _NBUF:4 * _NBUF]
        isem = bufs_and_sems[4 * _NBUF]

        wid = lax.axis_index("s") * nc + lax.axis_index("c")
        base_row = wid * bpw
        base_el = wid * epw

        pltpu.async_copy(
            idx_hbm.at[pl.ds(base_row, bpw)], idx_v, isem
        ).wait()

        def fire_gathers(g, b):
            for j in range(gpc):
                pltpu.async_copy(
                    tpad_hbm.at[idx_v.at[pl.ds(g * hist + j * _GR, _GR)]],
                    gbuf[b].at[pl.ds(j * _GR, _GR)],
                    gsem[b],
                )

        def drain_gathers(b):
            pltpu.make_async_copy(
                tpad_hbm.at[pl.ds(0, hist)], gbuf[b], gsem[b]
            ).wait()

        def compact(b):
            def rbody(r, carry):
                for c in range(_DIM // 16):
                    cbuf[b][r, pl.ds(c * 16, 16)] = gbuf[b][r, pl.ds(c * 16, 16)]
                return carry

            lax.fori_loop(0, hist, rbody, 0)

        def fire_writeback(g, b):
            el = base_el + g
            pltpu.async_copy(cbuf[b], out_hbm.at[el], wsem[b])

        def wait_writeback(b):
            pltpu.make_async_copy(cbuf[b], out_hbm.at[0], wsem[b]).wait()

        for b in range(_NBUF):
            fire_gathers(b, b)

        def body(s, carry):
            for b in range(_NBUF):
                g = s * _NBUF + b
                drain_gathers(b)
                compact(b)
                fire_writeback(g, b)
                wait_writeback(b)
                fire_gathers(g + _NBUF, b)
            return carry

        lax.fori_loop(0, nouter - 1, body, 0)

        for b in range(_NBUF):
            g = (nouter - 1) * _NBUF + b
            drain_gathers(b)
            compact(b)
            fire_writeback(g, b)
        for b in range(_NBUF):
            wait_writeback(b)

    return k


def kernel(to_embed, table):
    batch, hist = to_embed.shape
    idx = to_embed.reshape(-1).astype(jnp.int32)
    table_pad = jnp.pad(table, ((0, 0), (0, 128 - _DIM)))
    return _make_kernel(batch, hist)(idx, table_pad)
